# Initial kernel scaffold; baseline (speedup 1.0000x reference)
import jax
import jax.numpy as jnp
from jax.experimental import pallas as pl


def kernel(node_attr, edge_attr, edge_index, input_r, params):
    raise NotImplementedError("write your pallas kernel here")


# speedup vs baseline: 1.8590x; 1.8590x over previous
import functools

import jax
import jax.numpy as jnp
from jax import lax
from jax.experimental import pallas as pl
from jax.experimental.pallas import tpu as pltpu
from jax.experimental.pallas import tpu_sc as plsc

N_NODES = 10000
N_EDGES = 160000
LAT = 64

NC = 2
NS = 16
NW = NC * NS
EPW = N_EDGES // NW
CH = 1000
NCH = EPW // CH
RPT = N_NODES // NS

_f32 = jnp.float32


def _dot(a, b):
    return jnp.dot(a, b, preferred_element_type=_f32)


def _full(shape):
    return pl.BlockSpec(shape, lambda i: (0,) * len(shape))


def _prep_mlp(mlp):
    out = []
    for w, b in mlp:
        out.append(w)
        out.append(b.reshape(1, -1))
    return out


def _mlp3_body(x, w1, b1, w2, b2, w3, b3):
    h = jnp.maximum(_dot(x, w1) + b1, 0.0)
    h = jnp.maximum(_dot(h, w2) + b2, 0.0)
    return _dot(h, w3) + b3


def _enc_body(a, w1, b1, w2, b2, w3, b3, g, be, o):
    y = _mlp3_body(a[...], w1[...], b1[...], w2[...], b2[...], w3[...], b3[...])
    mu = jnp.mean(y, axis=-1, keepdims=True)
    var = jnp.mean((y - mu) ** 2, axis=-1, keepdims=True)
    o[...] = (y - mu) * lax.rsqrt(var + 1e-5) * g[...] + be[...]


def _encoder(a, mlp, ln, blk):
    n, fin = a.shape
    ws = _prep_mlp(mlp) + [ln[0].reshape(1, -1), ln[1].reshape(1, -1)]
    return pl.pallas_call(
        _enc_body,
        grid=(n // blk,),
        in_specs=[pl.BlockSpec((blk, fin), lambda i: (i, 0))]
        + [_full(w.shape) for w in ws],
        out_specs=pl.BlockSpec((blk, LAT), lambda i: (i, 0)),
        out_shape=jax.ShapeDtypeStruct((n, LAT), _f32),
    )(a, *ws)


def _edge_body(ef, xs, xd, w1e, w1s, w1d, b1, w2, b2, w3, b3, o):
    h = (_dot(ef[...], w1e[...]) + _dot(xs[...], w1s[...])
         + _dot(xd[...], w1d[...]) + b1[...])
    h = jnp.maximum(h, 0.0)
    h = jnp.maximum(_dot(h, w2[...]) + b2[...], 0.0)
    o[...] = _dot(h, w3[...]) + b3[...]


def _edge_mlp(ef, xs, xd, mlp, blk):
    (w1, b1), (w2, b2), (w3, b3) = mlp
    ws = [w1[:LAT], w1[LAT:2 * LAT], w1[2 * LAT:], b1.reshape(1, -1),
          w2, b2.reshape(1, -1), w3, b3.reshape(1, -1)]
    espec = pl.BlockSpec((blk, LAT), lambda i: (i, 0))
    return pl.pallas_call(
        _edge_body,
        grid=(N_EDGES // blk,),
        in_specs=[espec, espec, espec] + [_full(w.shape) for w in ws],
        out_specs=espec,
        out_shape=jax.ShapeDtypeStruct((N_EDGES, LAT), _f32),
    )(ef, xs, xd, *ws)


def _node_body(pa, pb, x, w1a, w1x, b1, w2, b2, w3, b3, o):
    aggr = pa[...] + pb[...]
    h = _dot(aggr, w1a[...]) + _dot(x[...], w1x[...]) + b1[...]
    h = jnp.maximum(h, 0.0)
    h = jnp.maximum(_dot(h, w2[...]) + b2[...], 0.0)
    o[...] = _dot(h, w3[...]) + b3[...]


def _node_mlp(pa, pb, x, mlp, blk):
    (w1, b1), (w2, b2), (w3, b3) = mlp
    ws = [w1[:LAT], w1[LAT:], b1.reshape(1, -1),
          w2, b2.reshape(1, -1), w3, b3.reshape(1, -1)]
    nspec = pl.BlockSpec((blk, LAT), lambda i: (i, 0))
    return pl.pallas_call(
        _node_body,
        grid=(N_NODES // blk,),
        in_specs=[nspec, nspec, nspec] + [_full(w.shape) for w in ws],
        out_specs=nspec,
        out_shape=jax.ShapeDtypeStruct((N_NODES, LAT), _f32),
    )(pa, pb, x, *ws)


def _head_body(x, w1, b1, w2, b2, w3, b3, o):
    o[...] = _mlp3_body(x[...], w1[...], b1[...], w2[...], b2[...], w3[...],
                        b3[...])


def _mlp_head(x, mlp, blk):
    n = x.shape[0]
    out_dim = mlp[-1][0].shape[1]
    ws = _prep_mlp(mlp)
    return pl.pallas_call(
        _head_body,
        grid=(n // blk,),
        in_specs=[pl.BlockSpec((blk, LAT), lambda i: (i, 0))]
        + [_full(w.shape) for w in ws],
        out_specs=pl.BlockSpec((blk, out_dim), lambda i: (i, 0)),
        out_shape=jax.ShapeDtypeStruct((n, out_dim), _f32),
    )(x, *ws)


def _sc_mesh():
    return plsc.VectorSubcoreMesh(core_axis_name="c", subcore_axis_name="s")


_SC_PARAMS = pltpu.CompilerParams(use_tc_tiling_on_sc=False)


def _sc_gather(table, src_idx, dst_idx):
    @functools.partial(
        pl.kernel,
        out_type=(jax.ShapeDtypeStruct((N_EDGES, LAT), _f32),
                  jax.ShapeDtypeStruct((N_EDGES, LAT), _f32)),
        mesh=_sc_mesh(),
        scratch_types=[
            pltpu.VMEM((CH,), jnp.int32),
            pltpu.VMEM((CH, LAT), _f32),
            pltpu.SemaphoreType.DMA,
        ],
        compiler_params=_SC_PARAMS,
    )
    def k(table_hbm, src_hbm, dst_hbm, xs_hbm, xd_hbm, idx_v, rows_v, sem):
        wid = lax.axis_index("s") * NC + lax.axis_index("c")
        base = wid * EPW

        def run(idx_hbm, out_hbm):
            def body(j, carry):
                off = base + j * CH
                pltpu.sync_copy(idx_hbm.at[pl.ds(off, CH)], idx_v)
                pltpu.async_copy(table_hbm.at[idx_v], rows_v, sem).wait()
                pltpu.sync_copy(rows_v, out_hbm.at[pl.ds(off, CH)])
                return carry
            lax.fori_loop(0, NCH, body, 0)

        run(src_hbm, xs_hbm)
        run(dst_hbm, xd_hbm)

    return k(table, src_idx, dst_idx)


def _sc_scatter_add(ef_new, dst_idx, zeros_nodes):
    @functools.partial(
        pl.kernel,
        out_type=jax.ShapeDtypeStruct((NC * N_NODES, LAT), _f32),
        mesh=_sc_mesh(),
        scratch_types=[
            pltpu.VMEM((CH,), jnp.int32),
            pltpu.VMEM((CH, LAT), _f32),
            pltpu.VMEM_SHARED((N_NODES, LAT), _f32),
        ],
        compiler_params=_SC_PARAMS,
    )
    def k(ef_hbm, dst_hbm, z_hbm, out_hbm, idx_v, rows_v, acc_sh):
        c = lax.axis_index("c")
        s = lax.axis_index("s")
        wid = s * NC + c
        base = wid * EPW
        r0 = s * RPT
        pltpu.sync_copy(z_hbm.at[pl.ds(r0, RPT)], acc_sh.at[pl.ds(r0, RPT)])
        plsc.subcore_barrier()

        def body(j, carry):
            off = base + j * CH
            pltpu.sync_copy(dst_hbm.at[pl.ds(off, CH)], idx_v)
            pltpu.sync_copy(ef_hbm.at[pl.ds(off, CH)], rows_v)
            pltpu.sync_copy(rows_v, acc_sh.at[idx_v], add=True)
            return carry
        lax.fori_loop(0, NCH, body, 0)

        plsc.subcore_barrier()
        pltpu.sync_copy(acc_sh.at[pl.ds(r0, RPT)],
                        out_hbm.at[pl.ds(c * N_NODES + r0, RPT)])

    return k(ef_new, dst_idx, zeros_nodes)


def kernel(node_attr, edge_attr, edge_index, input_r, params):
    x = _encoder(node_attr, params['enc_node']['mlp'],
                 params['enc_node']['ln'], blk=1000)
    ef = _encoder(edge_attr, params['enc_edge']['mlp'],
                  params['enc_edge']['ln'], blk=2000)

    perm = jnp.argsort(edge_index[0] * N_NODES + edge_index[1])
    src = jnp.take(edge_index[0], perm)
    dst = jnp.take(edge_index[1], perm)

    zeros_nodes = jnp.zeros((N_NODES, LAT), _f32)
    for lp in params['mp']:
        xs, xd = _sc_gather(x, src, dst)
        ef = _edge_mlp(ef, xs, xd, lp['edge'], blk=2000)
        partials = _sc_scatter_add(ef, dst, zeros_nodes)
        x = _node_mlp(partials[:N_NODES], partials[N_NODES:], x,
                      lp['node'], blk=1000)

    decoded_x = _mlp_head(x, params['dec_x'], blk=1000)
    decoded_L = _mlp_head(ef, params['dec_L'], blk=2000)
    return decoded_x, decoded_L
